# baseline (speedup 1.0000x reference)
import jax
import jax.numpy as jnp
from jax.experimental import pallas as pl

KNN = 12


def _offsets():
    r = jnp.array([-1.0, 0.0, 1.0], dtype=jnp.float32)
    oi, oj, ok = jnp.meshgrid(r, r, r, indexing='ij')
    return jnp.stack([oi.ravel(), oj.ravel(), ok.ravel()], axis=-1)


def _edges(cell, x, knn):
    b = cell.shape[0]
    n = x.shape[0]
    a = n // b
    xb = x.reshape(b, a, 3)
    offs = _offsets()
    xj_frac = xb[:, :, None, :] + offs[None, None, :, :]

    def rbf(v):
        u = jax.lax.bitcast_convert_type(v, jnp.uint32)
        lsb = (u >> 16) & jnp.uint32(1)
        u = (u + jnp.uint32(0x7FFF) + lsb) & jnp.uint32(0xFFFF0000)
        return jax.lax.bitcast_convert_type(u, jnp.float32)

    def mm3(u, c):
        u = rbf(u)
        c = rbf(c)
        return ((u[..., 0:1] * c[..., 0, :] + u[..., 1:2] * c[..., 1, :])
                + u[..., 2:3] * c[..., 2, :])

    xj_cart = mm3(xj_frac, cell[:, None, None]).reshape(b, a * 27, 3)
    xi_cart = mm3(xb, cell[:, None])
    nj = (xj_cart[..., 0] * xj_cart[..., 0] + xj_cart[..., 1] * xj_cart[..., 1]) + xj_cart[..., 2] * xj_cart[..., 2]
    ni = (xi_cart[..., 0] * xi_cart[..., 0] + xi_cart[..., 1] * xi_cart[..., 1]) + xi_cart[..., 2] * xi_cart[..., 2]
    xi_b = rbf(xi_cart)
    xj_b = rbf(xj_cart)
    dot = ((xi_b[:, :, None, 0] * xj_b[:, None, :, 0]
            + xi_b[:, :, None, 1] * xj_b[:, None, :, 1])
           + xi_b[:, :, None, 2] * xj_b[:, None, :, 2])
    d2 = (ni[:, :, None] + nj[:, None, :]) - 2.0 * dot
    self_mask = (jnp.eye(a, dtype=jnp.float32)[:, :, None]
                 * jax.nn.one_hot(13, 27, dtype=jnp.float32)[None, None, :]).reshape(a, a * 27)
    d2 = d2 + self_mask[None, :, :] * 1e12
    _, idx = jax.lax.top_k(-d2, knn)
    j = idx // 27
    o = idx % 27
    b_idx = jnp.arange(b)[:, None, None]
    i_idx = jnp.arange(a)[None, :, None]
    src = jnp.broadcast_to(b_idx * a + i_idx, j.shape).reshape(-1)
    dst = (b_idx * a + j).reshape(-1)
    cell_off = offs[o.reshape(-1)]
    return src, dst, cell_off


def _identity_body(x_ref, o_ref):
    o_ref[...] = x_ref[...]


def kernel(cell, x, x_tilde, num_atoms):
    src, dst, cell_off = _edges(cell, x, KNN)
    e_ij = x[dst] + cell_off - x[src]
    e_tilde_ij = x_tilde[dst] + cell_off - x_tilde[src]
    diff = jnp.abs(e_tilde_ij - e_ij)
    diff = pl.pallas_call(
        _identity_body,
        out_shape=jax.ShapeDtypeStruct(diff.shape, diff.dtype),
    )(diff)
    loss = jnp.mean(diff)
    return (loss, diff)


# speedup vs baseline: 21.7005x; 21.7005x over previous
import functools

import jax
import jax.numpy as jnp
from jax.experimental import pallas as pl
from jax.experimental.pallas import tpu as pltpu

KNN = 12
NOFF = 27

_OFFS = [(float(u), float(v), float(w))
         for u in (-1.0, 0.0, 1.0)
         for v in (-1.0, 0.0, 1.0)
         for w in (-1.0, 0.0, 1.0)]


def _rbf(v):
    u = jax.lax.bitcast_convert_type(v, jnp.uint32)
    lsb = (u >> 16) & jnp.uint32(1)
    u = (u + jnp.uint32(0x7FFF) + lsb) & jnp.uint32(0xFFFF0000)
    return jax.lax.bitcast_convert_type(u, jnp.float32)


def _body(cell_ref, xr_ref, xc_ref, xtr_ref, xtc_ref,
          diff_ref, loss_ref, d2_scr, acc_scr, *, a, b_total):
    bi = pl.program_id(0)
    cb = _rbf(cell_ref[...].reshape(3, 3))
    CB = [[cb[i:i + 1, j:j + 1] for j in range(3)] for i in range(3)]
    xR = xr_ref[...].reshape(3, a)
    xtR = xtr_ref[...].reshape(3, a)
    xC = xc_ref[...].reshape(a, 3)
    xtC = xtc_ref[...].reshape(a, 3)
    xrow = [xR[c:c + 1, :] for c in range(3)]
    xtrow = [xtR[c:c + 1, :] for c in range(3)]
    xcol = [xC[:, c:c + 1] for c in range(3)]
    xtcol = [xtC[:, c:c + 1] for c in range(3)]

    ub = [_rbf(xcol[c]) for c in range(3)]
    xi = [(ub[0] * CB[0][d] + ub[1] * CB[1][d]) + ub[2] * CB[2][d]
          for d in range(3)]
    ni = (xi[0] * xi[0] + xi[1] * xi[1]) + xi[2] * xi[2]
    xib = [_rbf(xi[d]) for d in range(3)]

    ii = jax.lax.broadcasted_iota(jnp.int32, (a, a), 0)
    jj = jax.lax.broadcasted_iota(jnp.int32, (a, a), 1)
    eye12 = jnp.where(ii == jj, jnp.float32(1e12), jnp.float32(0.0))

    for o in range(NOFF):
        off = _OFFS[o]
        uj = [_rbf(xrow[c] + off[c]) for c in range(3)]
        xj = [(uj[0] * CB[0][d] + uj[1] * CB[1][d]) + uj[2] * CB[2][d]
              for d in range(3)]
        nj = (xj[0] * xj[0] + xj[1] * xj[1]) + xj[2] * xj[2]
        xjb = [_rbf(xj[d]) for d in range(3)]
        dot = (xib[0] * xjb[0] + xib[1] * xjb[1]) + xib[2] * xjb[2]
        d2s = (ni + nj) - 2.0 * dot
        if o == 13:
            d2s = d2s + eye12
        d2_scr[:, o * a:(o + 1) * a] = d2s

    lane = jax.lax.broadcasted_iota(jnp.int32, (1, a * NOFF), 1)
    jrow = ((lane % a) * NOFF + lane // a).astype(jnp.float32)

    big = jnp.float32(3.0e38)
    sels = []
    for _ in range(KNN):
        d2v = d2_scr[...]
        m = jnp.min(d2v, axis=1, keepdims=True)
        sel = jnp.min(jnp.where(d2v == m, jrow, big), axis=1, keepdims=True)
        sels.append(sel)
        d2_scr[...] = jnp.where(jrow == sel, big, d2v)

    lane_f = jax.lax.broadcasted_iota(jnp.int32, (1, a), 1).astype(jnp.float32)
    cols = []
    for k in range(KNN):
        sel = sels[k]
        j = jnp.floor(sel / 27.0)
        o = sel - 27.0 * j
        q0 = jnp.floor(o / 9.0)
        r0 = o - 9.0 * q0
        q1 = jnp.floor(r0 / 3.0)
        q2 = r0 - 3.0 * q1
        offk = [q0 - 1.0, q1 - 1.0, q2 - 1.0]
        hit = lane_f == j
        for c in range(3):
            gx = jnp.sum(jnp.where(hit, xrow[c], 0.0), axis=1, keepdims=True)
            gxt = jnp.sum(jnp.where(hit, xtrow[c], 0.0), axis=1, keepdims=True)
            e = (gx + offk[c]) - xcol[c]
            et = (gxt + offk[c]) - xtcol[c]
            cols.append(jnp.abs(et - e))
    diffb = jnp.concatenate(cols, axis=1)
    diff_ref[...] = diffb.reshape(1, a, 3 * KNN)

    s = jnp.sum(jnp.sum(diffb, axis=1, keepdims=True), axis=0, keepdims=True)
    prev = jnp.where(bi == 0, jnp.zeros((1, 1), jnp.float32), acc_scr[...])
    tot = prev + s
    acc_scr[...] = tot
    loss_ref[...] = tot / jnp.float32(b_total * a * KNN * 3)


def kernel(cell, x, x_tilde, num_atoms):
    b = cell.shape[0]
    n = x.shape[0]
    a = n // b
    xb = x.reshape(b, a, 3)
    xtb = x_tilde.reshape(b, a, 3)
    xr = jnp.swapaxes(xb, 1, 2)
    xtr = jnp.swapaxes(xtb, 1, 2)
    body = functools.partial(_body, a=a, b_total=b)
    diff4, loss = pl.pallas_call(
        body,
        grid=(b,),
        in_specs=[
            pl.BlockSpec((1, 3, 3), lambda i: (i, 0, 0)),
            pl.BlockSpec((1, 3, a), lambda i: (i, 0, 0)),
            pl.BlockSpec((1, a, 3), lambda i: (i, 0, 0)),
            pl.BlockSpec((1, 3, a), lambda i: (i, 0, 0)),
            pl.BlockSpec((1, a, 3), lambda i: (i, 0, 0)),
        ],
        out_specs=(
            pl.BlockSpec((1, a, 3 * KNN), lambda i: (i, 0, 0)),
            pl.BlockSpec((1, 1), lambda i: (0, 0)),
        ),
        scratch_shapes=[
            pltpu.VMEM((a, a * NOFF), jnp.float32),
            pltpu.VMEM((1, 1), jnp.float32),
        ],
        out_shape=(
            jax.ShapeDtypeStruct((b, a, 3 * KNN), jnp.float32),
            jax.ShapeDtypeStruct((1, 1), jnp.float32),
        ),
    )(cell, xr, xb, xtr, xtb)
    return (loss.reshape(()), diff4.reshape(n * KNN, 3))
